# baseline (speedup 1.0000x reference)
import numpy as np

import jax
import jax.numpy as jnp
from jax.experimental import pallas as pl
from jax.experimental.pallas import tpu as pltpu
from jax.sharding import Mesh, PartitionSpec as P

W = 8
CHUNK = 8
NS_EPS = 1e-7


def _dot(a, b, dims):
    return jax.lax.dot_general(a, b, (dims, ((), ())),
                               preferred_element_type=jnp.float32)


def _atlas_kernel(x_ref, k_ref, v_ref, Mp_ref, Sp_ref, pc_ref,
                  aW_ref, ab_ref, eW_ref, eb_ref, tW_ref, tb_ref,
                  gW_ref, gb_ref, oW_ref, ob_ref,
                  out_ref, M_ref, S_ref,
                  bk_s, bv_s):
    pair = x_ref.shape[0]
    c = pl.program_id(0)

    @pl.when(c == 0)
    def _init():
        M_ref[...] = Mp_ref[...]
        S_ref[...] = Sp_ref[...]
        bk_s[...] = jnp.zeros_like(bk_s)
        bv_s[...] = jnp.zeros_like(bv_s)

    kphi_p, ks_p, vs_p, aT_p, eT_p, tT_p = [], [], [], [], [], []
    for i in range(pair):
        x_c = x_ref[i]
        k_c = k_ref[i]
        v_c = v_ref[i]
        kphi = pc_ref[0:1, :] * k_c + pc_ref[1:2, :] * (k_c * k_c)
        aT_p.append(jax.nn.sigmoid(_dot(aW_ref[...], x_c, ((1,), (1,))) + ab_ref[...]))
        eT_p.append(jax.nn.sigmoid(_dot(eW_ref[...], x_c, ((1,), (1,))) + eb_ref[...]) * 0.1)
        tT_p.append(jax.nn.sigmoid(_dot(tW_ref[...], x_c, ((1,), (1,))) + tb_ref[...]))
        g = jax.nn.sigmoid(_dot(x_c, gW_ref[...], ((1,), (0,))) + gb_ref[...])
        sg = jnp.sqrt(g)
        kphi_p.append(kphi)
        ks_p.append(sg * kphi)
        vs_p.append(sg * v_c)

    M_p = [M_ref[i] for i in range(pair)]
    S_p = [S_ref[i] for i in range(pair)]
    ys_p = [[] for _ in range(pair)]
    for j in range(CHUNK):
        rc = jnp.where(c == 0, jnp.float32(1.0 / (j + 1)), jnp.float32(1.0 / W))
        for i in range(pair):
            bk_s[i, j:j + 1, :] = ks_p[i][j:j + 1, :]
            bv_s[i, j:j + 1, :] = vs_p[i][j:j + 1, :]
            Kw = bk_s[i]
            Vw = bv_s[i]
            M, S = M_p[i], S_p[i]
            pe = (_dot(Kw, M, ((1,), (1,))) - Vw) * rc
            grad = _dot(pe, Kw, ((0,), (0,)))
            S = tT_p[i][:, j:j + 1] * S + grad
            nrm = jnp.sqrt(jnp.sum(S * S)) + NS_EPS
            SSt = _dot(S, S, ((1,), (1,)))
            SStS = _dot(SSt, S, ((1,), (0,)))
            ca = (1.5 / nrm) * eT_p[i][:, j:j + 1]
            cb = (0.5 / (nrm * nrm * nrm)) * eT_p[i][:, j:j + 1]
            M = aT_p[i][:, j:j + 1] * M - ca * S + cb * SStS
            M_p[i], S_p[i] = M, S
            ys_p[i].append(_dot(kphi_p[i][j:j + 1, :], M, ((1,), (1,))))

    for i in range(pair):
        Y = jnp.concatenate(ys_p[i], axis=0)
        out_ref[i] = _dot(Y, oW_ref[...], ((1,), (1,))) + ob_ref[...]
        M_ref[i] = M_p[i]
        S_ref[i] = S_p[i]


def _run_shard(x, k_aligned, v, M_prev, S_prev, poly_coeffs,
               alpha_W, alpha_b, eta_W, eta_b, theta_W, theta_b,
               gamma_W, gamma_b, out_W, out_b):
    Bl, L, D = x.shape
    nc = L // CHUNK

    row = lambda: pl.BlockSpec((Bl, CHUNK, D), lambda c: (0, c, 0))
    bat = lambda: pl.BlockSpec((Bl, D, D), lambda c: (0, 0, 0))
    fix = lambda s: pl.BlockSpec(s, lambda c: (0,) * len(s))

    return pl.pallas_call(
        _atlas_kernel,
        grid=(nc,),
        in_specs=[
            row(),
            row(),
            row(),
            bat(),
            bat(),
            fix((2, D)),
            fix((D, D)), fix((D, 1)),
            fix((D, D)), fix((D, 1)),
            fix((D, D)), fix((D, 1)),
            fix((D, 1)), fix((1, 1)),
            fix((D, D)), fix((1, D)),
        ],
        out_specs=[
            row(),
            bat(),
            bat(),
        ],
        out_shape=[
            jax.ShapeDtypeStruct((Bl, L, D), jnp.float32),
            jax.ShapeDtypeStruct((Bl, D, D), jnp.float32),
            jax.ShapeDtypeStruct((Bl, D, D), jnp.float32),
        ],
        scratch_shapes=[
            pltpu.VMEM((Bl, W, D), jnp.float32),
            pltpu.VMEM((Bl, W, D), jnp.float32),
        ],
        compiler_params=pltpu.CompilerParams(
            dimension_semantics=("arbitrary",),
        ),
        name="atlas_memory",
    )(x, k_aligned, v, M_prev, S_prev, poly_coeffs,
      alpha_W, alpha_b, eta_W, eta_b, theta_W, theta_b,
      gamma_W, gamma_b, out_W, out_b)


@jax.jit
def kernel(x, k_aligned, v, M_prev, S_prev, poly_coeffs,
           alpha_W, alpha_b, eta_W, eta_b, theta_W, theta_b,
           gamma_W, gamma_b, out_W, out_b):
    B, L, D = x.shape
    devs = jax.devices()
    nd = 2 if (len(devs) >= 2 and B % 2 == 0) else 1
    mesh = Mesh(np.asarray(devs[:nd]), ("x",))
    sharded = P("x")
    repl = P()
    fn = jax.shard_map(
        _run_shard, mesh=mesh,
        in_specs=(sharded, sharded, sharded, sharded, sharded,
                  repl, repl, repl, repl, repl, repl, repl,
                  repl, repl, repl, repl),
        out_specs=(sharded, sharded, sharded),
        check_vma=False,
    )
    out, M_out, S_out = fn(
        x, k_aligned, v, M_prev, S_prev, poly_coeffs,
        alpha_W, alpha_b.reshape(D, 1), eta_W, eta_b.reshape(D, 1),
        theta_W, theta_b.reshape(D, 1), gamma_W.reshape(D, 1),
        gamma_b.reshape(1, 1), out_W, out_b.reshape(1, D))
    return (out, M_out, S_out)
